# baseline (speedup 1.0000x reference)
import jax
import jax.numpy as jnp
import numpy as np
from jax.experimental import pallas as pl
from jax.experimental.pallas import tpu as pltpu

_N_LEVELS = 16
_T = 1 << 19
_BASE = 16
_PRIME_Y = 2654435761
_WIDTH = 128
_ENC = 32
_NOUT = 3
_NPTS = 2097152
_BN = 1024
_NB = _NPTS // _BN
_NTIL = _BN // 8

_LVL = []
_off = 0
for _l in range(_N_LEVELS):
    _res = _BASE << _l
    _dense = _res * _res <= _T
    _n = _res * _res if _dense else _T
    _LVL.append((_dense, _res, _off))
    _off += _n * 2 // 128
_RTOT = _off
_RPAD = ((_RTOT + 7) // 8) * 8


def _body(x_ref, y_ref, tab_ref, w8_ref, whid_ref, wout_ref, out_ref,
          iv_ref, stage_ref, smem_ref, sem_ref):
    x = x_ref[0]
    y = y_ref[0]
    lane = jax.lax.broadcasted_iota(jnp.int32, (8, 128), 1)
    pair_lane = lane >> 1

    def level_vectors(l):
        dense, res, off = _LVL[l]
        scale = float(res - 1)
        px = x * scale + 0.5
        py = y * scale + 0.5
        ix = px.astype(jnp.int32)
        iy = py.astype(jnp.int32)
        wx = px - ix.astype(jnp.float32)
        wy = py - iy.astype(jnp.float32)
        rows = []
        cvs = []
        ws = []
        for dx in (0, 1):
            for dy in (0, 1):
                cx = ix + dx
                cy = iy + dy
                if dense:
                    cx = jnp.minimum(cx, res - 1)
                    cy = jnp.minimum(cy, res - 1)
                    idx = cx + cy * res
                else:
                    hy = cy * jnp.int32(np.uint32(_PRIME_Y).view(np.int32))
                    idx = (cx ^ hy) & (_T - 1)
                rows.append((idx >> 6) + off)
                cvs.append(2 * (idx & 63))
                wxd = wx if dx else 1.0 - wx
                wyd = wy if dy else 1.0 - wy
                ws.append(wxd * wyd)
        return jnp.stack(rows, axis=0), cvs, ws

    def start_copy(l, slot):
        rows, cvs, ws = level_vectors(l)
        iv_ref[slot] = rows
        pltpu.make_async_copy(iv_ref.at[slot], smem_ref.at[slot],
                              sem_ref.at[slot]).start()
        return cvs, ws

    def gather_corner(l, c, slot, cv):
        def make_body(t_off):
            def body(i, pc):
                for u in range(2):
                    t = 2 * i + u + t_off
                    for s in range(8):
                        r = smem_ref[slot, c, s, t]
                        stage_ref[u, s] = tab_ref[r, 0]
                    stg = stage_ref[u, :, :]
                    tloc = t - t_off
                    ct = jnp.take_along_axis(
                        cv, jnp.full((8, 128), t, jnp.int32), axis=1)
                    idxv = (ct + lane - 2 * tloc) & 127
                    vals = jnp.take_along_axis(stg, idxv, axis=1)
                    pc = jnp.where(pair_lane == tloc, vals, pc)
                return pc
            return body
        zero = jnp.zeros((8, 128), jnp.float32)
        pcA = jax.lax.fori_loop(0, 32, make_body(0), zero)
        pcB = jax.lax.fori_loop(0, 32, make_body(64), zero)
        return pcA, pcB

    accs = []
    cvs, ws = start_copy(0, 0)
    for l in range(_N_LEVELS):
        slot = l & 1
        if l + 1 < _N_LEVELS:
            nxt = start_copy(l + 1, (l + 1) & 1)
        pltpu.make_async_copy(iv_ref.at[slot], smem_ref.at[slot],
                              sem_ref.at[slot]).wait()
        acc0 = jnp.zeros((8, 128), jnp.float32)
        acc1 = jnp.zeros((8, 128), jnp.float32)
        for c in range(4):
            pcA, pcB = gather_corner(l, c, slot, cvs[c])
            ev = (2 * lane) & 127
            od = (2 * lane + 1) & 127
            half = lane < 64
            f0 = jnp.where(half, jnp.take_along_axis(pcA, ev, axis=1),
                           jnp.take_along_axis(pcB, ev, axis=1))
            f1 = jnp.where(half, jnp.take_along_axis(pcA, od, axis=1),
                           jnp.take_along_axis(pcB, od, axis=1))
            acc0 = acc0 + ws[c] * f0
            acc1 = acc1 + ws[c] * f1
        accs.append(acc0)
        accs.append(acc1)
        if l + 1 < _N_LEVELS:
            cvs, ws = nxt

    pv = jnp.concatenate(accs, axis=0).astype(jnp.bfloat16)
    for s in range(8):
        h = jax.lax.dot_general(pv, w8_ref[s], (((0,), (0,)), ((), ())),
                                preferred_element_type=jnp.float32)
        h = jnp.maximum(h, 0.0).astype(jnp.bfloat16)
        for i in range(7):
            h = jax.lax.dot_general(h, whid_ref[i], (((1,), (0,)), ((), ())),
                                    preferred_element_type=jnp.float32)
            h = jnp.maximum(h, 0.0).astype(jnp.bfloat16)
        o = jax.lax.dot_general(h, wout_ref[...], (((1,), (0,)), ((), ())),
                                preferred_element_type=jnp.float32)
        out_ref[0, :, 8 * s:8 * s + 8] = o


def kernel(coords, table, w_in, w_hid, w_out):
    n_pts = coords.shape[0]
    nb = n_pts // _BN
    parts = []
    for l in range(_N_LEVELS):
        dense, res, off = _LVL[l]
        n = res * res if dense else _T
        parts.append(table[l, :n].reshape(-1, 128))
    tab = jnp.concatenate(parts, axis=0)
    tab = jnp.pad(tab, ((0, _RPAD - _RTOT), (0, 0))).reshape(_RPAD, 1, 128)

    xb = coords[:, 0].reshape(nb, 128, 8).transpose(0, 2, 1)
    yb = coords[:, 1].reshape(nb, 128, 8).transpose(0, 2, 1)

    eye = jnp.asarray(np.eye(8, dtype=np.float32))
    w8 = w_in[None, :, None, :] * eye[:, None, :, None]
    w8 = w8.reshape(8, _ENC * 8, _WIDTH).astype(jnp.bfloat16)
    whid = w_hid.astype(jnp.bfloat16)
    wout = jnp.pad(w_out, ((0, 0), (0, 5))).astype(jnp.bfloat16)

    out = pl.pallas_call(
        _body,
        out_shape=jax.ShapeDtypeStruct((nb, 128, 64), jnp.float32),
        grid=(nb,),
        in_specs=[
            pl.BlockSpec((1, 8, 128), lambda i: (i, 0, 0)),
            pl.BlockSpec((1, 8, 128), lambda i: (i, 0, 0)),
            pl.BlockSpec((_RPAD, 1, 128), lambda i: (0, 0, 0)),
            pl.BlockSpec((8, _ENC * 8, _WIDTH), lambda i: (0, 0, 0)),
            pl.BlockSpec((_N_LEVELS // 2 - 1, _WIDTH, _WIDTH), lambda i: (0, 0, 0)),
            pl.BlockSpec((_WIDTH, 8), lambda i: (0, 0)),
        ],
        out_specs=pl.BlockSpec((1, 128, 64), lambda i: (i, 0, 0)),
        scratch_shapes=[
            pltpu.VMEM((2, 4, 8, 128), jnp.int32),
            pltpu.VMEM((2, 8, 128), jnp.float32),
            pltpu.SMEM((2, 4, 8, 128), jnp.int32),
            pltpu.SemaphoreType.DMA((2,)),
        ],
        compiler_params=pltpu.CompilerParams(
            dimension_semantics=("parallel",),
        ),
        name="tiny_inr_fused",
    )(xb, yb, tab, w8, whid, wout)
    return out.reshape(n_pts, 8)[:, :_NOUT]


# speedup vs baseline: 3.8484x; 3.8484x over previous
import jax
import jax.numpy as jnp
import numpy as np
from jax.experimental import pallas as pl
from jax.experimental.pallas import tpu as pltpu

_N_LEVELS = 16
_T = 1 << 19
_BASE = 16
_PRIME_Y = 2654435761
_WIDTH = 128
_ENC = 32
_NOUT = 3
_NPTS = 2097152
_BN = 1024
_NB = _NPTS // _BN
_NTIL = _BN // 8

_LVL = []
_off = 0
for _l in range(_N_LEVELS):
    _res = _BASE << _l
    _dense = _res * _res <= _T
    _n = _res * _res if _dense else _T
    _LVL.append((_dense, _res, _off))
    _off += _n * 2 // 128
_RTOT = _off
_RPAD = ((_RTOT + 7) // 8) * 8


def _body(x_ref, y_ref, tab_ref, w8_ref, whid_ref, wout_ref, out_ref,
          iv_ref, stage_a, stage_b, smem_ref, sem_ref):
    x = x_ref[0]
    y = y_ref[0]
    lane = jax.lax.broadcasted_iota(jnp.int32, (8, 128), 1)
    pair_lane = lane >> 1

    def level_vectors(l):
        dense, res, off = _LVL[l]
        scale = float(res - 1)
        px = x * scale + 0.5
        py = y * scale + 0.5
        ix = px.astype(jnp.int32)
        iy = py.astype(jnp.int32)
        wx = px - ix.astype(jnp.float32)
        wy = py - iy.astype(jnp.float32)
        rows = []
        cvs = []
        ws = []
        for dx in (0, 1):
            for dy in (0, 1):
                cx = ix + dx
                cy = iy + dy
                if dense:
                    cx = jnp.minimum(cx, res - 1)
                    cy = jnp.minimum(cy, res - 1)
                    idx = cx + cy * res
                else:
                    hy = cy * jnp.int32(np.uint32(_PRIME_Y).view(np.int32))
                    idx = (cx ^ hy) & (_T - 1)
                rows.append((idx >> 6) + off)
                cvs.append(2 * (idx & 63))
                wxd = wx if dx else 1.0 - wx
                wyd = wy if dy else 1.0 - wy
                ws.append(wxd * wyd)
        return jnp.stack(rows, axis=0), cvs, ws

    def start_copy(l, slot):
        rows, cvs, ws = level_vectors(l)
        iv_ref[slot] = rows
        pltpu.make_async_copy(iv_ref.at[slot], smem_ref.at[slot],
                              sem_ref.at[slot]).start()
        return cvs, ws

    t3 = jax.lax.broadcasted_iota(jnp.int32, (128, 8, 128), 0)
    l3 = jax.lax.broadcasted_iota(jnp.int32, (128, 8, 128), 2)
    shift3 = l3 - 2 * t3
    mask3 = (l3 >> 1) == (t3 & 63)

    def gather_corner(l, c, slot, cv, stg_ref):
        def gbody(i, carry):
            for u in range(8):
                t = 8 * i + u
                for s in range(8):
                    r = smem_ref[slot, c, s, t]
                    stg_ref[t, s] = tab_ref[r, 0]
            return carry
        jax.lax.fori_loop(0, 16, gbody, 0)
        cv3 = jnp.broadcast_to(cv[None, :, :], (128, 8, 128))
        b3 = jnp.take_along_axis(cv3, t3, axis=2)
        idxm = (b3 + shift3) & 127
        vals = jnp.take_along_axis(stg_ref[...], idxm, axis=2)
        vz = jnp.where(mask3, vals, 0.0)
        pcA = jnp.sum(vz[:64], axis=0)
        pcB = jnp.sum(vz[64:], axis=0)
        return pcA, pcB

    accs = []
    cvs, ws = start_copy(0, 0)
    for l in range(_N_LEVELS):
        slot = l & 1
        if l + 1 < _N_LEVELS:
            nxt = start_copy(l + 1, (l + 1) & 1)
        pltpu.make_async_copy(iv_ref.at[slot], smem_ref.at[slot],
                              sem_ref.at[slot]).wait()
        acc0 = jnp.zeros((8, 128), jnp.float32)
        acc1 = jnp.zeros((8, 128), jnp.float32)
        for c in range(4):
            stg_ref = stage_a if c & 1 else stage_b
            pcA, pcB = gather_corner(l, c, slot, cvs[c], stg_ref)
            ev = (2 * lane) & 127
            od = (2 * lane + 1) & 127
            half = lane < 64
            f0 = jnp.where(half, jnp.take_along_axis(pcA, ev, axis=1),
                           jnp.take_along_axis(pcB, ev, axis=1))
            f1 = jnp.where(half, jnp.take_along_axis(pcA, od, axis=1),
                           jnp.take_along_axis(pcB, od, axis=1))
            acc0 = acc0 + ws[c] * f0
            acc1 = acc1 + ws[c] * f1
        accs.append(acc0)
        accs.append(acc1)
        if l + 1 < _N_LEVELS:
            cvs, ws = nxt

    pv = jnp.concatenate(accs, axis=0).astype(jnp.bfloat16)
    for s in range(8):
        h = jax.lax.dot_general(pv, w8_ref[s], (((0,), (0,)), ((), ())),
                                preferred_element_type=jnp.float32)
        h = jnp.maximum(h, 0.0).astype(jnp.bfloat16)
        for i in range(7):
            h = jax.lax.dot_general(h, whid_ref[i], (((1,), (0,)), ((), ())),
                                    preferred_element_type=jnp.float32)
            h = jnp.maximum(h, 0.0).astype(jnp.bfloat16)
        o = jax.lax.dot_general(h, wout_ref[...], (((1,), (0,)), ((), ())),
                                preferred_element_type=jnp.float32)
        out_ref[0, :, 8 * s:8 * s + 8] = o


def kernel(coords, table, w_in, w_hid, w_out):
    n_pts = coords.shape[0]
    nb = n_pts // _BN
    parts = []
    for l in range(_N_LEVELS):
        dense, res, off = _LVL[l]
        n = res * res if dense else _T
        parts.append(table[l, :n].reshape(-1, 128))
    tab = jnp.concatenate(parts, axis=0)
    tab = jnp.pad(tab, ((0, _RPAD - _RTOT), (0, 0))).reshape(_RPAD, 1, 128)

    xb = coords[:, 0].reshape(nb, 128, 8).transpose(0, 2, 1)
    yb = coords[:, 1].reshape(nb, 128, 8).transpose(0, 2, 1)

    eye = jnp.asarray(np.eye(8, dtype=np.float32))
    w8 = w_in[None, :, None, :] * eye[:, None, :, None]
    w8 = w8.reshape(8, _ENC * 8, _WIDTH).astype(jnp.bfloat16)
    whid = w_hid.astype(jnp.bfloat16)
    wout = jnp.pad(w_out, ((0, 0), (0, 5))).astype(jnp.bfloat16)

    out = pl.pallas_call(
        _body,
        out_shape=jax.ShapeDtypeStruct((nb, 128, 64), jnp.float32),
        grid=(nb,),
        in_specs=[
            pl.BlockSpec((1, 8, 128), lambda i: (i, 0, 0)),
            pl.BlockSpec((1, 8, 128), lambda i: (i, 0, 0)),
            pl.BlockSpec((_RPAD, 1, 128), lambda i: (0, 0, 0)),
            pl.BlockSpec((8, _ENC * 8, _WIDTH), lambda i: (0, 0, 0)),
            pl.BlockSpec((_N_LEVELS // 2 - 1, _WIDTH, _WIDTH), lambda i: (0, 0, 0)),
            pl.BlockSpec((_WIDTH, 8), lambda i: (0, 0)),
        ],
        out_specs=pl.BlockSpec((1, 128, 64), lambda i: (i, 0, 0)),
        scratch_shapes=[
            pltpu.VMEM((2, 4, 8, 128), jnp.int32),
            pltpu.VMEM((128, 8, 128), jnp.float32),
            pltpu.VMEM((128, 8, 128), jnp.float32),
            pltpu.SMEM((2, 4, 8, 128), jnp.int32),
            pltpu.SemaphoreType.DMA((2,)),
        ],
        compiler_params=pltpu.CompilerParams(
            dimension_semantics=("parallel",),
        ),
        name="tiny_inr_fused",
    )(xb, yb, tab, w8, whid, wout)
    return out.reshape(n_pts, 8)[:, :_NOUT]


# speedup vs baseline: 3.9071x; 1.0153x over previous
import jax
import jax.numpy as jnp
import numpy as np
from jax.experimental import pallas as pl
from jax.experimental.pallas import tpu as pltpu

_N_LEVELS = 16
_T = 1 << 19
_BASE = 16
_PRIME_Y = 2654435761
_WIDTH = 128
_ENC = 32
_NOUT = 3
_NPTS = 2097152
_BN = 1024
_NB = _NPTS // _BN
_NTIL = _BN // 8

_LVL = []
_off = 0
for _l in range(_N_LEVELS):
    _res = _BASE << _l
    _dense = _res * _res <= _T
    _n = _res * _res if _dense else _T
    _LVL.append((_dense, _res, _off))
    _off += _n * 2 // 128
_RTOT = _off
_RPAD = ((_RTOT + 7) // 8) * 8


def _body(x_ref, y_ref, tab_ref, w8_ref, whid_ref, wout_ref, out_ref,
          iv_ref, stage_a, stage_b, smem_ref, sem_ref):
    x = x_ref[0]
    y = y_ref[0]
    lane = jax.lax.broadcasted_iota(jnp.int32, (8, 128), 1)
    pair_lane = lane >> 1

    def level_vectors(l):
        dense, res, off = _LVL[l]
        scale = float(res - 1)
        px = x * scale + 0.5
        py = y * scale + 0.5
        ix = px.astype(jnp.int32)
        iy = py.astype(jnp.int32)
        wx = px - ix.astype(jnp.float32)
        wy = py - iy.astype(jnp.float32)
        rows = []
        cvs = []
        ws = []
        for dx in (0, 1):
            for dy in (0, 1):
                cx = ix + dx
                cy = iy + dy
                if dense:
                    cx = jnp.minimum(cx, res - 1)
                    cy = jnp.minimum(cy, res - 1)
                    idx = cx + cy * res
                else:
                    hy = cy * jnp.int32(np.uint32(_PRIME_Y).view(np.int32))
                    idx = (cx ^ hy) & (_T - 1)
                rows.append((idx >> 6) + off)
                cvs.append(2 * (idx & 63))
                wxd = wx if dx else 1.0 - wx
                wyd = wy if dy else 1.0 - wy
                ws.append(wxd * wyd)
        return jnp.stack(rows, axis=0), cvs, ws

    def start_copy(l, slot):
        rows, cvs, ws = level_vectors(l)
        iv_ref[slot] = rows
        pltpu.make_async_copy(iv_ref.at[slot], smem_ref.at[slot],
                              sem_ref.at[slot]).start()
        return cvs, ws

    t3 = jax.lax.broadcasted_iota(jnp.int32, (128, 8, 128), 0)
    l3 = jax.lax.broadcasted_iota(jnp.int32, (128, 8, 128), 2)
    shift3 = l3 - 2 * t3
    mask3 = (l3 >> 1) == (t3 & 63)

    def gather_corner(l, c, slot, cv, stg_ref):
        def gbody(i, carry):
            for u in range(16):
                t = 16 * i + u
                for s in range(8):
                    r = smem_ref[slot, c, s, t]
                    stg_ref[t, s] = tab_ref[r, 0]
            return carry
        jax.lax.fori_loop(0, 8, gbody, 0)
        cv3 = jnp.broadcast_to(cv[None, :, :], (128, 8, 128))
        b3 = jnp.take_along_axis(cv3, t3, axis=2)
        idxm = (b3 + shift3) & 127
        vals = jnp.take_along_axis(stg_ref[...], idxm, axis=2)
        vz = jnp.where(mask3, vals, 0.0)
        pcA = jnp.sum(vz[:64], axis=0)
        pcB = jnp.sum(vz[64:], axis=0)
        return pcA, pcB

    accs = []
    cvs, ws = start_copy(0, 0)
    for l in range(_N_LEVELS):
        slot = l & 1
        if l + 1 < _N_LEVELS:
            nxt = start_copy(l + 1, (l + 1) & 1)
        pltpu.make_async_copy(iv_ref.at[slot], smem_ref.at[slot],
                              sem_ref.at[slot]).wait()
        acc0 = jnp.zeros((8, 128), jnp.float32)
        acc1 = jnp.zeros((8, 128), jnp.float32)
        for c in range(4):
            stg_ref = stage_a if c & 1 else stage_b
            pcA, pcB = gather_corner(l, c, slot, cvs[c], stg_ref)
            ev = (2 * lane) & 127
            od = (2 * lane + 1) & 127
            half = lane < 64
            f0 = jnp.where(half, jnp.take_along_axis(pcA, ev, axis=1),
                           jnp.take_along_axis(pcB, ev, axis=1))
            f1 = jnp.where(half, jnp.take_along_axis(pcA, od, axis=1),
                           jnp.take_along_axis(pcB, od, axis=1))
            acc0 = acc0 + ws[c] * f0
            acc1 = acc1 + ws[c] * f1
        accs.append(acc0)
        accs.append(acc1)
        if l + 1 < _N_LEVELS:
            cvs, ws = nxt

    pv = jnp.concatenate(accs, axis=0).astype(jnp.bfloat16)
    for s in range(8):
        h = jax.lax.dot_general(pv, w8_ref[s], (((0,), (0,)), ((), ())),
                                preferred_element_type=jnp.float32)
        h = jnp.maximum(h, 0.0).astype(jnp.bfloat16)
        for i in range(7):
            h = jax.lax.dot_general(h, whid_ref[i], (((1,), (0,)), ((), ())),
                                    preferred_element_type=jnp.float32)
            h = jnp.maximum(h, 0.0).astype(jnp.bfloat16)
        o = jax.lax.dot_general(h, wout_ref[...], (((1,), (0,)), ((), ())),
                                preferred_element_type=jnp.float32)
        out_ref[0, :, 8 * s:8 * s + 8] = o


def kernel(coords, table, w_in, w_hid, w_out):
    n_pts = coords.shape[0]
    nb = n_pts // _BN
    parts = []
    for l in range(_N_LEVELS):
        dense, res, off = _LVL[l]
        n = res * res if dense else _T
        parts.append(table[l, :n].reshape(-1, 128))
    tab = jnp.concatenate(parts, axis=0)
    tab = jnp.pad(tab, ((0, _RPAD - _RTOT), (0, 0))).reshape(_RPAD, 1, 128)

    xb = coords[:, 0].reshape(nb, 128, 8).transpose(0, 2, 1)
    yb = coords[:, 1].reshape(nb, 128, 8).transpose(0, 2, 1)

    eye = jnp.asarray(np.eye(8, dtype=np.float32))
    w8 = w_in[None, :, None, :] * eye[:, None, :, None]
    w8 = w8.reshape(8, _ENC * 8, _WIDTH).astype(jnp.bfloat16)
    whid = w_hid.astype(jnp.bfloat16)
    wout = jnp.pad(w_out, ((0, 0), (0, 5))).astype(jnp.bfloat16)

    out = pl.pallas_call(
        _body,
        out_shape=jax.ShapeDtypeStruct((nb, 128, 64), jnp.float32),
        grid=(nb,),
        in_specs=[
            pl.BlockSpec((1, 8, 128), lambda i: (i, 0, 0)),
            pl.BlockSpec((1, 8, 128), lambda i: (i, 0, 0)),
            pl.BlockSpec((_RPAD, 1, 128), lambda i: (0, 0, 0)),
            pl.BlockSpec((8, _ENC * 8, _WIDTH), lambda i: (0, 0, 0)),
            pl.BlockSpec((_N_LEVELS // 2 - 1, _WIDTH, _WIDTH), lambda i: (0, 0, 0)),
            pl.BlockSpec((_WIDTH, 8), lambda i: (0, 0)),
        ],
        out_specs=pl.BlockSpec((1, 128, 64), lambda i: (i, 0, 0)),
        scratch_shapes=[
            pltpu.VMEM((2, 4, 8, 128), jnp.int32),
            pltpu.VMEM((128, 8, 128), jnp.float32),
            pltpu.VMEM((128, 8, 128), jnp.float32),
            pltpu.SMEM((2, 4, 8, 128), jnp.int32),
            pltpu.SemaphoreType.DMA((2,)),
        ],
        compiler_params=pltpu.CompilerParams(
            dimension_semantics=("parallel",),
        ),
        name="tiny_inr_fused",
    )(xb, yb, tab, w8, whid, wout)
    return out.reshape(n_pts, 8)[:, :_NOUT]


# speedup vs baseline: 4.4636x; 1.1424x over previous
import jax
import jax.numpy as jnp
import numpy as np
from jax.experimental import pallas as pl
from jax.experimental.pallas import tpu as pltpu

_N_LEVELS = 16
_T = 1 << 19
_BASE = 16
_PRIME_Y = 2654435761
_WIDTH = 128
_ENC = 32
_NOUT = 3
_NPTS = 2097152
_BN = 1024
_NB = _NPTS // _BN
_NTIL = _BN // 8

_LVL = []
_off = 0
for _l in range(_N_LEVELS):
    _res = _BASE << _l
    _dense = _res * _res <= _T
    _LVL.append((_dense, _res, _off))
    _off += (_res * _res * 4 if _dense else _T * 2) // 128
_RTOT = _off
_RPAD = ((_RTOT + 7) // 8) * 8


def _body(x_ref, y_ref, tab_ref, w8_ref, whid_ref, wout_ref, out_ref,
          iv_ref, stage_a, stage_b, smem_ref, sem_ref):
    x = x_ref[0]
    y = y_ref[0]
    lane = jax.lax.broadcasted_iota(jnp.int32, (8, 128), 1)
    pair_lane = lane >> 1

    def level_vectors(l):
        dense, res, off = _LVL[l]
        scale = float(res - 1)
        px = x * scale + 0.5
        py = y * scale + 0.5
        ix = px.astype(jnp.int32)
        iy = py.astype(jnp.int32)
        wx = px - ix.astype(jnp.float32)
        wy = py - iy.astype(jnp.float32)
        units = []
        if dense:
            rows = []
            for dy in (0, 1):
                cy = jnp.minimum(iy + dy, res - 1)
                cell = ix + cy * res
                rows.append((cell >> 5) + off)
                cvq = 4 * (cell & 31)
                wyd = wy if dy else 1.0 - wy
                units.append((dy, cvq, (1.0 - wx) * wyd))
                units.append((dy, cvq + 2, wx * wyd))
            return jnp.stack(rows, axis=0), units
        rows = []
        for dx in (0, 1):
            for dy in (0, 1):
                cx = ix + dx
                cy = iy + dy
                hy = cy * jnp.int32(np.uint32(_PRIME_Y).view(np.int32))
                idx = (cx ^ hy) & (_T - 1)
                rows.append((idx >> 6) + off)
                wxd = wx if dx else 1.0 - wx
                wyd = wy if dy else 1.0 - wy
                units.append((len(rows) - 1, 2 * (idx & 63), wxd * wyd))
        return jnp.stack(rows, axis=0), units

    def start_copy(l, slot):
        rows, units = level_vectors(l)
        ng = rows.shape[0]
        iv_ref[slot, :ng] = rows
        pltpu.make_async_copy(iv_ref.at[slot, :ng], smem_ref.at[slot, :ng],
                              sem_ref.at[slot]).start()
        return units

    t3 = jax.lax.broadcasted_iota(jnp.int32, (128, 8, 128), 0)
    l3 = jax.lax.broadcasted_iota(jnp.int32, (128, 8, 128), 2)
    shift3 = l3 - 2 * t3
    mask3 = (l3 >> 1) == (t3 & 63)

    def gather_into(g, slot, stg_ref):
        def gbody(i, carry):
            for u in range(16):
                t = 16 * i + u
                for s in range(8):
                    r = smem_ref[slot, g, s, t]
                    stg_ref[t, s] = tab_ref[r, 0]
            return carry
        jax.lax.fori_loop(0, 8, gbody, 0)

    def extract(cv, stg_ref):
        cv3 = jnp.broadcast_to(cv[None, :, :], (128, 8, 128))
        b3 = jnp.take_along_axis(cv3, t3, axis=2)
        idxm = (b3 + shift3) & 127
        vals = jnp.take_along_axis(stg_ref[...], idxm, axis=2)
        vz = jnp.where(mask3, vals, 0.0)
        pcA = jnp.sum(vz[:64], axis=0)
        pcB = jnp.sum(vz[64:], axis=0)
        return pcA, pcB

    accs = []
    units = start_copy(0, 0)
    for l in range(_N_LEVELS):
        slot = l & 1
        if l + 1 < _N_LEVELS:
            nxt = start_copy(l + 1, (l + 1) & 1)
        ng = 2 if _LVL[l][0] else 4
        pltpu.make_async_copy(iv_ref.at[slot, :ng], smem_ref.at[slot, :ng],
                              sem_ref.at[slot]).wait()
        acc0 = jnp.zeros((8, 128), jnp.float32)
        acc1 = jnp.zeros((8, 128), jnp.float32)
        last_g = -1
        for g, cv, w in units:
            stg_ref = stage_a if g & 1 else stage_b
            if g != last_g:
                gather_into(g, slot, stg_ref)
                last_g = g
            pcA, pcB = extract(cv, stg_ref)
            ev = (2 * lane) & 127
            od = (2 * lane + 1) & 127
            half = lane < 64
            f0 = jnp.where(half, jnp.take_along_axis(pcA, ev, axis=1),
                           jnp.take_along_axis(pcB, ev, axis=1))
            f1 = jnp.where(half, jnp.take_along_axis(pcA, od, axis=1),
                           jnp.take_along_axis(pcB, od, axis=1))
            acc0 = acc0 + w * f0
            acc1 = acc1 + w * f1
        accs.append(acc0)
        accs.append(acc1)
        if l + 1 < _N_LEVELS:
            units = nxt

    pv = jnp.concatenate(accs, axis=0).astype(jnp.bfloat16)
    for s in range(8):
        h = jax.lax.dot_general(pv, w8_ref[s], (((0,), (0,)), ((), ())),
                                preferred_element_type=jnp.float32)
        h = jnp.maximum(h, 0.0).astype(jnp.bfloat16)
        for i in range(7):
            h = jax.lax.dot_general(h, whid_ref[i], (((1,), (0,)), ((), ())),
                                    preferred_element_type=jnp.float32)
            h = jnp.maximum(h, 0.0).astype(jnp.bfloat16)
        o = jax.lax.dot_general(h, wout_ref[...], (((1,), (0,)), ((), ())),
                                preferred_element_type=jnp.float32)
        out_ref[0, :, 8 * s:8 * s + 8] = o


def kernel(coords, table, w_in, w_hid, w_out):
    n_pts = coords.shape[0]
    nb = n_pts // _BN
    parts = []
    for l in range(_N_LEVELS):
        dense, res, off = _LVL[l]
        if dense:
            t2 = table[l, :res * res].reshape(res, res, 2)
            t2x = jnp.concatenate([t2[:, 1:], t2[:, -1:]], axis=1)
            parts.append(jnp.concatenate([t2, t2x], axis=2).reshape(-1, 128))
        else:
            parts.append(table[l].reshape(-1, 128))
    tab = jnp.concatenate(parts, axis=0)
    tab = jnp.pad(tab, ((0, _RPAD - _RTOT), (0, 0))).reshape(_RPAD, 1, 128)

    xb = coords[:, 0].reshape(nb, 128, 8).transpose(0, 2, 1)
    yb = coords[:, 1].reshape(nb, 128, 8).transpose(0, 2, 1)

    eye = jnp.asarray(np.eye(8, dtype=np.float32))
    w8 = w_in[None, :, None, :] * eye[:, None, :, None]
    w8 = w8.reshape(8, _ENC * 8, _WIDTH).astype(jnp.bfloat16)
    whid = w_hid.astype(jnp.bfloat16)
    wout = jnp.pad(w_out, ((0, 0), (0, 5))).astype(jnp.bfloat16)

    out = pl.pallas_call(
        _body,
        out_shape=jax.ShapeDtypeStruct((nb, 128, 64), jnp.float32),
        grid=(nb,),
        in_specs=[
            pl.BlockSpec((1, 8, 128), lambda i: (i, 0, 0)),
            pl.BlockSpec((1, 8, 128), lambda i: (i, 0, 0)),
            pl.BlockSpec((_RPAD, 1, 128), lambda i: (0, 0, 0)),
            pl.BlockSpec((8, _ENC * 8, _WIDTH), lambda i: (0, 0, 0)),
            pl.BlockSpec((_N_LEVELS // 2 - 1, _WIDTH, _WIDTH), lambda i: (0, 0, 0)),
            pl.BlockSpec((_WIDTH, 8), lambda i: (0, 0)),
        ],
        out_specs=pl.BlockSpec((1, 128, 64), lambda i: (i, 0, 0)),
        scratch_shapes=[
            pltpu.VMEM((2, 4, 8, 128), jnp.int32),
            pltpu.VMEM((128, 8, 128), jnp.float32),
            pltpu.VMEM((128, 8, 128), jnp.float32),
            pltpu.SMEM((2, 4, 8, 128), jnp.int32),
            pltpu.SemaphoreType.DMA((2,)),
        ],
        compiler_params=pltpu.CompilerParams(
            dimension_semantics=("parallel",),
        ),
        name="tiny_inr_fused",
    )(xb, yb, tab, w8, whid, wout)
    return out.reshape(n_pts, 8)[:, :_NOUT]
